# baseline (speedup 1.0000x reference)
import functools

import jax
import jax.numpy as jnp
from jax import lax
from jax.experimental import pallas as pl
from jax.experimental.pallas import tpu as pltpu
from jax.experimental.pallas import tpu_sc as plsc

_B = 4
_N = 8192
_D = 256
_L = 64
_NPAIR = _L * _L
_CHUNK = 1024
_NCH = _N // _CHUNK
_LCHUNK = 2048
_NW = 32
_PTS_PER_TILE = _B * _N // _NW


_BPG = 2


def _select_witness_body(cur0_ref, emb_ref, c_ref, keys_ref, wd_ref,
                         embT_ref, xn_ref, fac1_ref, mind_ref, lm_ref):
    g = pl.program_id(0)
    c = c_ref[0, 0]

    for k in range(_BPG):
        for ch in range(_NCH):
            x = emb_ref[k, pl.ds(ch * _CHUNK, _CHUNK), :]
            xt = jnp.transpose(x)
            embT_ref[:, pl.ds(k * _N + ch * _CHUNK, _CHUNK)] = (
                xt.astype(jnp.bfloat16))
            xn = jnp.sum(xt * xt, axis=0, keepdims=True)
            xn_ref[pl.ds(k, 1), pl.ds(ch * _CHUNK, _CHUNK)] = xn
            fac1_ref[pl.ds(k, 1), pl.ds(ch * _CHUNK, _CHUNK)] = 1.0 + c * xn

    mind_ref[...] = jnp.full((_BPG, _N), jnp.inf, jnp.float32)
    cur0s = []
    for k in range(_BPG):
        cur0 = cur0_ref[g * _BPG + k]
        lm_ref[k, pl.ds(0, 1), :] = emb_ref[k, pl.ds(cur0, 1), :]
        cur0s.append(cur0)

    def step(i, carry):
        ys = [emb_ref[k, pl.ds(carry[k], 1), :] for k in range(_BPG)]
        y2 = jnp.concatenate(ys, axis=0)
        out = jnp.dot(y2.astype(jnp.bfloat16), embT_ref[...],
                      preferred_element_type=jnp.float32)
        new = []
        for k in range(_BPG):
            yn = jnp.sum(ys[k] * ys[k])
            xy = out[k:k + 1, k * _N:(k + 1) * _N]
            diff = jnp.clip(xn_ref[pl.ds(k, 1), :] + yn - 2.0 * xy,
                            1e-10, None)
            fac = fac1_ref[pl.ds(k, 1), :] + c * yn
            d2 = diff * fac
            mind = jnp.minimum(mind_ref[pl.ds(k, 1), :], d2)
            mind_ref[pl.ds(k, 1), :] = mind
            amax = jnp.max(mind)
            pos = lax.broadcasted_iota(jnp.int32, (1, _N), 1)
            newcur = jnp.min(jnp.where(mind == amax, pos, _N))
            lm_ref[k, pl.ds(i, 1), :] = emb_ref[k, pl.ds(newcur, 1), :]
            new.append(newcur)
        return tuple(new)

    lax.fori_loop(1, _L, step, tuple(cur0s))

    for k in range(_BPG):
        lm = lm_ref[k]
        yn_col = jnp.sum(lm * lm, axis=1, keepdims=True)
        for chl in range(_N // _LCHUNK):
            sl = pl.ds(chl * _LCHUNK, _LCHUNK)
            embT_sl = embT_ref[:, pl.ds(k * _N + chl * _LCHUNK,
                                        _LCHUNK)]
            xy = jnp.dot(lm.astype(jnp.bfloat16), embT_sl,
                         preferred_element_type=jnp.float32)
            xn_sl = xn_ref[pl.ds(k, 1), sl]
            diff = jnp.clip(xn_sl + yn_col - 2.0 * xy, 1e-10, None)
            fac = jnp.clip(1.0 + c * xn_sl + c * yn_col, 1e-6, None)
            dl = jnp.sqrt(diff) * jnp.sqrt(fac)

            iota0 = lax.broadcasted_iota(jnp.int32, (_L, _LCHUNK), 0)
            m0 = jnp.min(dl, axis=0, keepdims=True)
            i0 = jnp.min(jnp.where(dl == m0, iota0, _L),
                         axis=0, keepdims=True)
            dmask = jnp.where(iota0 == i0, jnp.inf, dl)
            m1 = jnp.min(dmask, axis=0, keepdims=True)
            i1 = jnp.min(jnp.where(dmask == m1, iota0, _L),
                         axis=0, keepdims=True)
            ii = jnp.minimum(i0, i1)
            jj = jnp.maximum(i0, i1)
            keys_ref[k, :, sl] = ii * _L + jj
            wd_ref[k, :, sl] = m1


def _select_witness(cur0, emb, c_arr):
    grid_spec = pltpu.PrefetchScalarGridSpec(
        num_scalar_prefetch=1,
        grid=(_B // _BPG,),
        in_specs=[
            pl.BlockSpec((_BPG, _N, _D), lambda g, sref: (g, 0, 0)),
            pl.BlockSpec((8, 128), lambda g, sref: (0, 0)),
        ],
        out_specs=[
            pl.BlockSpec((_BPG, 1, _N), lambda g, sref: (g, 0, 0)),
            pl.BlockSpec((_BPG, 1, _N), lambda g, sref: (g, 0, 0)),
        ],
        scratch_shapes=[
            pltpu.VMEM((_D, _BPG * _N), jnp.bfloat16),
            pltpu.VMEM((_BPG, _N), jnp.float32),
            pltpu.VMEM((_BPG, _N), jnp.float32),
            pltpu.VMEM((_BPG, _N), jnp.float32),
            pltpu.VMEM((_BPG, _L, _D), jnp.float32),
        ],
    )
    return pl.pallas_call(
        _select_witness_body,
        grid_spec=grid_spec,
        out_shape=[
            jax.ShapeDtypeStruct((_B, 1, _N), jnp.int32),
            jax.ShapeDtypeStruct((_B, 1, _N), jnp.float32),
        ],
    )(cur0, emb, c_arr)


def _sc_scatter_body(keys_hbm, wd_hbm, min_hbm, cnt_hbm,
                     keys_v, wd_v, min_v, cnt_v):
    wid = lax.axis_index("s") * 2 + lax.axis_index("c")
    base = wid * _PTS_PER_TILE
    pltpu.sync_copy(keys_hbm.at[pl.ds(base, _PTS_PER_TILE)], keys_v)
    pltpu.sync_copy(wd_hbm.at[pl.ds(base, _PTS_PER_TILE)], wd_v)

    inf16 = jnp.full((16,), jnp.inf, jnp.float32)
    zero16 = jnp.zeros((16,), jnp.float32)

    def init_body(i, carry):
        min_v[pl.ds(i * 16, 16)] = inf16
        cnt_v[pl.ds(i * 16, 16)] = zero16
        return carry

    lax.fori_loop(0, _NPAIR // 16, init_body, 0)

    one16 = jnp.full((16,), 1.0, jnp.float32)

    def point_body(t, carry):
        k = keys_v[pl.ds(t * 16, 16)]
        v = wd_v[pl.ds(t * 16, 16)]
        plsc.addupdate_scatter(cnt_v, [k], one16)

        def cond(rem):
            return jnp.sum(rem) > 0

        def body(rem):
            cur = plsc.load_gather(min_v, [k])
            newv = jnp.minimum(cur, v)
            plsc.store_scatter(min_v, [k], newv, mask=rem > 0)
            chk = plsc.load_gather(min_v, [k])
            return jnp.where(chk <= v, 0, rem)

        lax.while_loop(cond, body, jnp.ones((16,), jnp.int32))
        return carry

    lax.fori_loop(0, _PTS_PER_TILE // 16, point_body, 0)

    pltpu.sync_copy(min_v, min_hbm.at[wid])
    pltpu.sync_copy(cnt_v, cnt_hbm.at[wid])


def _sc_scatter(keys_flat, wd_flat):
    mesh = plsc.VectorSubcoreMesh(core_axis_name="c", subcore_axis_name="s")
    f = functools.partial(
        pl.kernel,
        mesh=mesh,
        compiler_params=pltpu.CompilerParams(needs_layout_passes=False),
        out_type=[
            jax.ShapeDtypeStruct((_NW, _NPAIR), jnp.float32),
            jax.ShapeDtypeStruct((_NW, _NPAIR), jnp.float32),
        ],
        scratch_types=[
            pltpu.VMEM((_PTS_PER_TILE,), jnp.int32),
            pltpu.VMEM((_PTS_PER_TILE,), jnp.float32),
            pltpu.VMEM((_NPAIR,), jnp.float32),
            pltpu.VMEM((_NPAIR,), jnp.float32),
        ],
    )(_sc_scatter_body)
    return f(keys_flat, wd_flat)


def _finalize_body(min_ref, cnt_ref, ew_ref, wc_ref, betti_ref):
    mins = jnp.min(min_ref[...], axis=0)
    cnts = jnp.sum(cnt_ref[...], axis=0)
    ew_ref[...] = jnp.minimum(mins, mins.T)[None]
    wc_ref[...] = (cnts + cnts.T)[None]

    finite = mins < jnp.inf
    ri = lax.broadcasted_iota(jnp.int32, (_L, _L), 0)
    rj = lax.broadcasted_iota(jnp.int32, (_L, _L), 1)
    adj = jnp.logical_or(finite, finite.T)
    r = jnp.where(jnp.logical_or(adj, ri == rj), 1.0, 0.0)
    for _ in range(6):
        r = (jnp.dot(r, r, preferred_element_type=jnp.float32)
             > 0.0).astype(jnp.float32)
    lab = jnp.min(jnp.where(r > 0.0, rj, _L), axis=1, keepdims=True)
    iota_i = lax.broadcasted_iota(jnp.int32, (_L, 1), 0)
    comps = jnp.sum(jnp.where(lab == iota_i, 1, 0))
    nedges = jnp.sum(jnp.where(finite, 1, 0))
    loops = nedges - (_L - comps)
    bi = lax.broadcasted_iota(jnp.int32, (1, 2), 1)
    betti_ref[...] = jnp.where(bi == 0, comps.astype(jnp.float32),
                               loops.astype(jnp.float32))[None]


def _finalize(minacc, cntacc):
    tiles_per_batch = _NW // _B
    return pl.pallas_call(
        _finalize_body,
        grid=(_B,),
        in_specs=[
            pl.BlockSpec((tiles_per_batch, _L, _L), lambda b: (b, 0, 0)),
            pl.BlockSpec((tiles_per_batch, _L, _L), lambda b: (b, 0, 0)),
        ],
        out_specs=[
            pl.BlockSpec((1, _L, _L), lambda b: (b, 0, 0)),
            pl.BlockSpec((1, _L, _L), lambda b: (b, 0, 0)),
            pl.BlockSpec((1, 1, 2), lambda b: (b, 0, 0)),
        ],
        out_shape=[
            jax.ShapeDtypeStruct((_B, _L, _L), jnp.float32),
            jax.ShapeDtypeStruct((_B, _L, _L), jnp.float32),
            jax.ShapeDtypeStruct((_B, 1, 2), jnp.float32),
        ],
    )(minacc, cntacc)


def kernel(embeddings, curvature_scale):
    c = jnp.abs(curvature_scale).astype(jnp.float32)
    key = jax.random.key(42)
    cur0 = jax.random.randint(key, (_B,), 0, _N).astype(jnp.int32)
    c_arr = jnp.full((8, 128), c, jnp.float32)
    keys, wd = _select_witness(cur0, embeddings, c_arr)
    minacc, cntacc = _sc_scatter(keys.reshape(-1), wd.reshape(-1))
    ew, wc, betti = _finalize(minacc.reshape(_NW, _L, _L),
                              cntacc.reshape(_NW, _L, _L))
    return ew, wc, betti.reshape(_B, 2)


# speedup vs baseline: 1.0145x; 1.0145x over previous
import functools

import jax
import jax.numpy as jnp
from jax import lax
from jax.experimental import pallas as pl
from jax.experimental.pallas import tpu as pltpu
from jax.experimental.pallas import tpu_sc as plsc

_B = 4
_N = 8192
_D = 256
_L = 64
_NPAIR = _L * _L
_CHUNK = 1024
_NCH = _N // _CHUNK
_LCHUNK = 2048
_NW = 32
_PTS_PER_TILE = _B * _N // _NW


_BPG = 2


def _select_witness_body(cur0_ref, emb_ref, c_ref, keys_ref, wd_ref,
                         embT_ref, xn_ref, fac1_ref, mind_ref, lm_ref):
    g = pl.program_id(0)
    c = c_ref[0, 0]

    for k in range(_BPG):
        for ch in range(_NCH):
            x = emb_ref[k, pl.ds(ch * _CHUNK, _CHUNK), :]
            xt = jnp.transpose(x)
            embT_ref[:, pl.ds(k * _N + ch * _CHUNK, _CHUNK)] = (
                xt.astype(jnp.bfloat16))
            xn = jnp.sum(xt * xt, axis=0, keepdims=True)
            xn_ref[pl.ds(k, 1), pl.ds(ch * _CHUNK, _CHUNK)] = xn
            fac1_ref[pl.ds(k, 1), pl.ds(ch * _CHUNK, _CHUNK)] = 1.0 + c * xn

    mind_ref[...] = jnp.full((_BPG, _N), jnp.inf, jnp.float32)
    cur0s = []
    for k in range(_BPG):
        cur0 = cur0_ref[g * _BPG + k]
        lm_ref[k, pl.ds(0, 1), :] = emb_ref[k, pl.ds(cur0, 1), :]
        cur0s.append(cur0)

    def step(i, carry):
        ys = [emb_ref[k, pl.ds(carry[k], 1), :] for k in range(_BPG)]
        y2 = jnp.concatenate(ys, axis=0)
        out = jnp.dot(y2.astype(jnp.bfloat16), embT_ref[...],
                      preferred_element_type=jnp.float32)
        lane = lax.broadcasted_iota(jnp.int32, (1, 128), 1)
        new = []
        for k in range(_BPG):
            yn = jnp.sum(ys[k] * ys[k])
            cyn = c * yn
            cv = cp = None
            chains = []
            for r in range(_N // 128):
                sl = pl.ds(r * 128, 128)
                xy_r = out[k:k + 1, k * _N + r * 128:k * _N + (r + 1) * 128]
                diff = jnp.clip(xn_ref[pl.ds(k, 1), sl] + yn - 2.0 * xy_r,
                                1e-10, None)
                d2 = diff * (fac1_ref[pl.ds(k, 1), sl] + cyn)
                m = jnp.minimum(mind_ref[pl.ds(k, 1), sl], d2)
                mind_ref[pl.ds(k, 1), sl] = m
                if r % 8 == 0:
                    if cv is not None:
                        chains.append((cv, cp))
                    cv, cp = m, jnp.full((1, 128), r, jnp.int32)
                else:
                    gt = m > cv
                    cv = jnp.where(gt, m, cv)
                    cp = jnp.where(gt, r, cp)
            chains.append((cv, cp))
            bv, bp = chains[0]
            for qv, qp in chains[1:]:
                gt = qv > bv
                bv = jnp.where(gt, qv, bv)
                bp = jnp.where(gt, qp, bp)
            pos = bp * 128 + lane
            amax = jnp.max(bv)
            newcur = jnp.min(jnp.where(bv == amax, pos, _N))
            lm_ref[k, pl.ds(i, 1), :] = emb_ref[k, pl.ds(newcur, 1), :]
            new.append(newcur)
        return tuple(new)

    lax.fori_loop(1, _L, step, tuple(cur0s))

    for k in range(_BPG):
        lm = lm_ref[k]
        yn_col = jnp.sum(lm * lm, axis=1, keepdims=True)
        for chl in range(_N // _LCHUNK):
            sl = pl.ds(chl * _LCHUNK, _LCHUNK)
            embT_sl = embT_ref[:, pl.ds(k * _N + chl * _LCHUNK,
                                        _LCHUNK)]
            xy = jnp.dot(lm.astype(jnp.bfloat16), embT_sl,
                         preferred_element_type=jnp.float32)
            xn_sl = xn_ref[pl.ds(k, 1), sl]
            diff = jnp.clip(xn_sl + yn_col - 2.0 * xy, 1e-10, None)
            fac = jnp.clip(1.0 + c * xn_sl + c * yn_col, 1e-6, None)
            dl = jnp.sqrt(diff) * jnp.sqrt(fac)

            iota0 = lax.broadcasted_iota(jnp.int32, (_L, _LCHUNK), 0)
            m0 = jnp.min(dl, axis=0, keepdims=True)
            i0 = jnp.min(jnp.where(dl == m0, iota0, _L),
                         axis=0, keepdims=True)
            dmask = jnp.where(iota0 == i0, jnp.inf, dl)
            m1 = jnp.min(dmask, axis=0, keepdims=True)
            i1 = jnp.min(jnp.where(dmask == m1, iota0, _L),
                         axis=0, keepdims=True)
            ii = jnp.minimum(i0, i1)
            jj = jnp.maximum(i0, i1)
            keys_ref[k, :, sl] = ii * _L + jj
            wd_ref[k, :, sl] = m1


def _select_witness(cur0, emb, c_arr):
    grid_spec = pltpu.PrefetchScalarGridSpec(
        num_scalar_prefetch=1,
        grid=(_B // _BPG,),
        in_specs=[
            pl.BlockSpec((_BPG, _N, _D), lambda g, sref: (g, 0, 0)),
            pl.BlockSpec((8, 128), lambda g, sref: (0, 0)),
        ],
        out_specs=[
            pl.BlockSpec((_BPG, 1, _N), lambda g, sref: (g, 0, 0)),
            pl.BlockSpec((_BPG, 1, _N), lambda g, sref: (g, 0, 0)),
        ],
        scratch_shapes=[
            pltpu.VMEM((_D, _BPG * _N), jnp.bfloat16),
            pltpu.VMEM((_BPG, _N), jnp.float32),
            pltpu.VMEM((_BPG, _N), jnp.float32),
            pltpu.VMEM((_BPG, _N), jnp.float32),
            pltpu.VMEM((_BPG, _L, _D), jnp.float32),
        ],
    )
    return pl.pallas_call(
        _select_witness_body,
        grid_spec=grid_spec,
        out_shape=[
            jax.ShapeDtypeStruct((_B, 1, _N), jnp.int32),
            jax.ShapeDtypeStruct((_B, 1, _N), jnp.float32),
        ],
    )(cur0, emb, c_arr)


def _sc_scatter_body(keys_hbm, wd_hbm, min_hbm, cnt_hbm,
                     keys_v, wd_v, min_v, cnt_v):
    wid = lax.axis_index("s") * 2 + lax.axis_index("c")
    base = wid * _PTS_PER_TILE
    pltpu.sync_copy(keys_hbm.at[pl.ds(base, _PTS_PER_TILE)], keys_v)
    pltpu.sync_copy(wd_hbm.at[pl.ds(base, _PTS_PER_TILE)], wd_v)

    inf16 = jnp.full((16,), jnp.inf, jnp.float32)
    zero16 = jnp.zeros((16,), jnp.float32)

    def init_body(i, carry):
        min_v[pl.ds(i * 16, 16)] = inf16
        cnt_v[pl.ds(i * 16, 16)] = zero16
        return carry

    lax.fori_loop(0, _NPAIR // 16, init_body, 0)

    one16 = jnp.full((16,), 1.0, jnp.float32)

    def point_body(t, carry):
        k = keys_v[pl.ds(t * 16, 16)]
        v = wd_v[pl.ds(t * 16, 16)]
        plsc.addupdate_scatter(cnt_v, [k], one16)

        def cond(rem):
            return jnp.sum(rem) > 0

        def body(rem):
            cur = plsc.load_gather(min_v, [k])
            newv = jnp.minimum(cur, v)
            plsc.store_scatter(min_v, [k], newv, mask=rem > 0)
            chk = plsc.load_gather(min_v, [k])
            return jnp.where(chk <= v, 0, rem)

        lax.while_loop(cond, body, jnp.ones((16,), jnp.int32))
        return carry

    lax.fori_loop(0, _PTS_PER_TILE // 16, point_body, 0)

    pltpu.sync_copy(min_v, min_hbm.at[wid])
    pltpu.sync_copy(cnt_v, cnt_hbm.at[wid])


def _sc_scatter(keys_flat, wd_flat):
    mesh = plsc.VectorSubcoreMesh(core_axis_name="c", subcore_axis_name="s")
    f = functools.partial(
        pl.kernel,
        mesh=mesh,
        compiler_params=pltpu.CompilerParams(needs_layout_passes=False),
        out_type=[
            jax.ShapeDtypeStruct((_NW, _NPAIR), jnp.float32),
            jax.ShapeDtypeStruct((_NW, _NPAIR), jnp.float32),
        ],
        scratch_types=[
            pltpu.VMEM((_PTS_PER_TILE,), jnp.int32),
            pltpu.VMEM((_PTS_PER_TILE,), jnp.float32),
            pltpu.VMEM((_NPAIR,), jnp.float32),
            pltpu.VMEM((_NPAIR,), jnp.float32),
        ],
    )(_sc_scatter_body)
    return f(keys_flat, wd_flat)


def _finalize_body(min_ref, cnt_ref, ew_ref, wc_ref, betti_ref):
    mins = jnp.min(min_ref[...], axis=0)
    cnts = jnp.sum(cnt_ref[...], axis=0)
    ew_ref[...] = jnp.minimum(mins, mins.T)[None]
    wc_ref[...] = (cnts + cnts.T)[None]

    finite = mins < jnp.inf
    ri = lax.broadcasted_iota(jnp.int32, (_L, _L), 0)
    rj = lax.broadcasted_iota(jnp.int32, (_L, _L), 1)
    adj = jnp.logical_or(finite, finite.T)
    r = jnp.where(jnp.logical_or(adj, ri == rj), 1.0, 0.0)
    for _ in range(6):
        r = (jnp.dot(r, r, preferred_element_type=jnp.float32)
             > 0.0).astype(jnp.float32)
    lab = jnp.min(jnp.where(r > 0.0, rj, _L), axis=1, keepdims=True)
    iota_i = lax.broadcasted_iota(jnp.int32, (_L, 1), 0)
    comps = jnp.sum(jnp.where(lab == iota_i, 1, 0))
    nedges = jnp.sum(jnp.where(finite, 1, 0))
    loops = nedges - (_L - comps)
    bi = lax.broadcasted_iota(jnp.int32, (1, 2), 1)
    betti_ref[...] = jnp.where(bi == 0, comps.astype(jnp.float32),
                               loops.astype(jnp.float32))[None]


def _finalize(minacc, cntacc):
    tiles_per_batch = _NW // _B
    return pl.pallas_call(
        _finalize_body,
        grid=(_B,),
        in_specs=[
            pl.BlockSpec((tiles_per_batch, _L, _L), lambda b: (b, 0, 0)),
            pl.BlockSpec((tiles_per_batch, _L, _L), lambda b: (b, 0, 0)),
        ],
        out_specs=[
            pl.BlockSpec((1, _L, _L), lambda b: (b, 0, 0)),
            pl.BlockSpec((1, _L, _L), lambda b: (b, 0, 0)),
            pl.BlockSpec((1, 1, 2), lambda b: (b, 0, 0)),
        ],
        out_shape=[
            jax.ShapeDtypeStruct((_B, _L, _L), jnp.float32),
            jax.ShapeDtypeStruct((_B, _L, _L), jnp.float32),
            jax.ShapeDtypeStruct((_B, 1, 2), jnp.float32),
        ],
    )(minacc, cntacc)


def kernel(embeddings, curvature_scale):
    c = jnp.abs(curvature_scale).astype(jnp.float32)
    key = jax.random.key(42)
    cur0 = jax.random.randint(key, (_B,), 0, _N).astype(jnp.int32)
    c_arr = jnp.full((8, 128), c, jnp.float32)
    keys, wd = _select_witness(cur0, embeddings, c_arr)
    minacc, cntacc = _sc_scatter(keys.reshape(-1), wd.reshape(-1))
    ew, wc, betti = _finalize(minacc.reshape(_NW, _L, _L),
                              cntacc.reshape(_NW, _L, _L))
    return ew, wc, betti.reshape(_B, 2)
